# baseline (speedup 1.0000x reference)
import functools

import jax
import jax.numpy as jnp
from jax import lax
from jax.experimental import pallas as pl
from jax.experimental.pallas import tpu as pltpu
from jax.experimental.pallas import tpu_sc as plsc

CHUNK = 128
NG = 4
LAG = 2


@functools.lru_cache(maxsize=None)
def _build(n_i: int, n_j: int, d_embed: int, d_pad: int):
    info = plsc.get_sparse_core_info()
    nw = info.num_cores * info.num_subcores
    nc = info.num_cores
    i_per_w = n_i // nw
    m_per_j = i_per_w // CHUNK
    n_chunks = n_j * m_per_j

    mesh = plsc.VectorSubcoreMesh(core_axis_name="c", subcore_axis_name="s")

    @functools.partial(
        pl.kernel,
        mesh=mesh,
        out_type=jax.ShapeDtypeStruct((n_j, d_embed, n_i), jnp.float32),
        scratch_types=[
            pltpu.VMEM((n_j, i_per_w), jnp.int32),
            pltpu.VMEM((NG, CHUNK, d_pad), jnp.float32),
            pltpu.VMEM((2, d_embed, CHUNK), jnp.float32),
            pltpu.SemaphoreType.DMA,
            pltpu.SemaphoreType.DMA,
        ],
        compiler_params=pltpu.CompilerParams(
            use_tc_tiling_on_sc=True, needs_layout_passes=False
        ),
    )
    def k(xt_hbm, tpad_hbm, out_hbm, idx_v, g_v, t_v, gsem, osem):
        wid = lax.axis_index("s") * nc + lax.axis_index("c")
        i_base = wid * i_per_w
        pltpu.sync_copy(xt_hbm.at[:, pl.ds(i_base, i_per_w)], idx_v)

        def gather(c):
            j = c // m_per_j
            m = lax.rem(c, m_per_j)
            return pltpu.make_async_copy(
                tpad_hbm.at[idx_v.at[j, pl.ds(m * CHUNK, CHUNK)]],
                g_v.at[lax.rem(c, NG)],
                gsem,
            )

        def writeback(c, tb):
            j = c // m_per_j
            m = lax.rem(c, m_per_j)
            return pltpu.make_async_copy(
                t_v.at[tb],
                out_hbm.at[j, :, pl.ds(i_base + m * CHUNK, CHUNK)],
                osem,
            )

        lane = lax.iota(jnp.int32, 16)
        row_idx = [jnp.full((16,), 16 * kk, jnp.int32) + lane
                   for kk in range(CHUNK // 16)]

        for c in range(LAG):
            gather(c).start()

        def body(c, carry):
            @pl.when(c < n_chunks - LAG)
            def _():
                gather(c + LAG).start()

            @pl.when(c >= 2)
            def _():
                writeback(c - 2, lax.rem(c, 2)).wait()

            gather(c).wait()
            gb = lax.rem(c, NG)
            tb = lax.rem(c, 2)

            def trans_e(e, carry2):
                col = jnp.broadcast_to(e, (16,)).astype(jnp.int32)
                for kk in range(CHUNK // 16):
                    v = plsc.load_gather(g_v.at[gb], [row_idx[kk], col])
                    t_v[tb, e, pl.ds(16 * kk, 16)] = v
                return carry2

            lax.fori_loop(0, d_embed, trans_e, 0)
            writeback(c, tb).start()
            return carry

        lax.fori_loop(0, n_chunks, body, 0)
        writeback(n_chunks - 2, lax.rem(n_chunks - 2, 2)).wait()
        writeback(n_chunks - 1, lax.rem(n_chunks - 1, 2)).wait()

    return k


def kernel(x, table):
    n_i, n_j = x.shape
    d_embed = table.shape[1]
    d_pad = 128
    xt = x.T.astype(jnp.int32)
    tpad = jnp.pad(table, ((0, 0), (0, d_pad - d_embed)))
    out3 = _build(n_i, n_j, d_embed, d_pad)(xt, tpad)
    return jnp.transpose(out3, (2, 0, 1))


# speedup vs baseline: 1.1515x; 1.1515x over previous
import functools

import jax
import jax.numpy as jnp
from jax import lax
from jax.experimental import pallas as pl
from jax.experimental.pallas import tpu as pltpu
from jax.experimental.pallas import tpu_sc as plsc

CHUNK = 128
NG = 4
LAG = 2


@functools.lru_cache(maxsize=None)
def _build(n_i: int, n_j: int, d_embed: int, d_pad: int):
    info = plsc.get_sparse_core_info()
    nw = info.num_cores * info.num_subcores
    nc = info.num_cores
    i_per_w = n_i // nw
    m_per_j = i_per_w // CHUNK
    n_chunks = n_j * m_per_j

    mesh = plsc.VectorSubcoreMesh(core_axis_name="c", subcore_axis_name="s")

    @functools.partial(
        pl.kernel,
        mesh=mesh,
        out_type=jax.ShapeDtypeStruct((n_j, d_embed, n_i), jnp.float32),
        scratch_types=[
            pltpu.VMEM((n_j, i_per_w), jnp.int32),
            pltpu.VMEM((NG, CHUNK, d_pad), jnp.float32),
            pltpu.VMEM((2, d_embed, CHUNK), jnp.float32),
            pltpu.SemaphoreType.DMA,
            pltpu.SemaphoreType.DMA,
        ],
        compiler_params=pltpu.CompilerParams(
            use_tc_tiling_on_sc=True, needs_layout_passes=False
        ),
    )
    def k(xt_hbm, tpad_hbm, out_hbm, idx_v, g_v, t_v, gsem, osem):
        wid = lax.axis_index("s") * nc + lax.axis_index("c")
        i_base = wid * i_per_w
        pltpu.sync_copy(xt_hbm.at[:, pl.ds(i_base, i_per_w)], idx_v)

        def gather(c):
            j = c // m_per_j
            m = lax.rem(c, m_per_j)
            return pltpu.make_async_copy(
                tpad_hbm.at[idx_v.at[j, pl.ds(m * CHUNK, CHUNK)]],
                g_v.at[lax.rem(c, NG)],
                gsem,
            )

        def writeback(c, tb):
            j = c // m_per_j
            m = lax.rem(c, m_per_j)
            return pltpu.make_async_copy(
                t_v.at[tb],
                out_hbm.at[j, :, pl.ds(i_base + m * CHUNK, CHUNK)],
                osem,
            )

        lane = lax.iota(jnp.int32, 16)
        e_idx = [jnp.full((16,), 16 * kk, jnp.int32) + lane
                 for kk in range(d_embed // 16)]

        for c in range(LAG):
            gather(c).start()

        UNROLL = 4

        def body(c, carry):
            @pl.when(c < n_chunks - LAG)
            def _():
                gather(c + LAG).start()

            @pl.when(c >= 2)
            def _():
                writeback(c - 2, lax.rem(c, 2)).wait()

            gather(c).wait()
            gb = lax.rem(c, NG)
            tb = lax.rem(c, 2)
            t2d = t_v.at[tb]

            def trans_i(i0, carry2):
                for u in range(UNROLL):
                    i = i0 * UNROLL + u
                    i_bc = jnp.broadcast_to(i, (16,)).astype(jnp.int32)
                    for kk in range(d_embed // 16):
                        v = g_v[gb, i, pl.ds(16 * kk, 16)]
                        plsc.store_scatter(t2d, [e_idx[kk], i_bc], v)
                return carry2

            lax.fori_loop(0, CHUNK // UNROLL, trans_i, 0)
            writeback(c, tb).start()
            return carry

        lax.fori_loop(0, n_chunks, body, 0)
        writeback(n_chunks - 2, lax.rem(n_chunks - 2, 2)).wait()
        writeback(n_chunks - 1, lax.rem(n_chunks - 1, 2)).wait()

    return k


def kernel(x, table):
    n_i, n_j = x.shape
    d_embed = table.shape[1]
    d_pad = 128
    xt = x.T.astype(jnp.int32)
    tpad = jnp.pad(table, ((0, 0), (0, d_pad - d_embed)))
    out3 = _build(n_i, n_j, d_embed, d_pad)(xt, tpad)
    return jnp.transpose(out3, (2, 0, 1))


# speedup vs baseline: 1.1788x; 1.0237x over previous
import functools

import jax
import jax.numpy as jnp
from jax import lax
from jax.experimental import pallas as pl
from jax.experimental.pallas import tpu as pltpu
from jax.experimental.pallas import tpu_sc as plsc

CHUNK = 128
NG = 4
LAG = 2


@functools.lru_cache(maxsize=None)
def _build(n_i: int, n_j: int, d_embed: int, d_pad: int):
    info = plsc.get_sparse_core_info()
    nw = info.num_cores * info.num_subcores
    nc = info.num_cores
    i_per_w = n_i // nw
    m_per_j = i_per_w // CHUNK
    n_chunks = n_j * m_per_j

    mesh = plsc.VectorSubcoreMesh(core_axis_name="c", subcore_axis_name="s")

    @functools.partial(
        pl.kernel,
        mesh=mesh,
        out_type=jax.ShapeDtypeStruct((n_j, d_embed, n_i), jnp.float32),
        scratch_types=[
            pltpu.VMEM((n_j, i_per_w), jnp.int32),
            pltpu.VMEM((NG, CHUNK, d_pad), jnp.float32),
            pltpu.VMEM((2, d_embed, CHUNK), jnp.float32),
            pltpu.SemaphoreType.DMA,
            pltpu.SemaphoreType.DMA,
        ],
        compiler_params=pltpu.CompilerParams(
            use_tc_tiling_on_sc=True, needs_layout_passes=False
        ),
    )
    def k(xt_hbm, tpad_hbm, out_hbm, idx_v, g_v, t_v, gsem, osem):
        wid = lax.axis_index("s") * nc + lax.axis_index("c")
        i_base = wid * i_per_w
        pltpu.sync_copy(xt_hbm.at[:, pl.ds(i_base, i_per_w)], idx_v)

        def gather(c):
            j = c // m_per_j
            m = lax.rem(c, m_per_j)
            return pltpu.make_async_copy(
                tpad_hbm.at[idx_v.at[j, pl.ds(m * CHUNK, CHUNK)]],
                g_v.at[lax.rem(c, NG)],
                gsem,
            )

        def writeback(c, tb):
            j = c // m_per_j
            m = lax.rem(c, m_per_j)
            return pltpu.make_async_copy(
                t_v.at[tb],
                out_hbm.at[j, :, pl.ds(i_base + m * CHUNK, CHUNK)],
                osem,
            )

        lane = lax.iota(jnp.int32, 16)
        e_idx = [jnp.full((16,), 16 * kk, jnp.int32) + lane
                 for kk in range(d_embed // 16)]

        for c in range(LAG):
            gather(c).start()

        UNROLL = 4

        def body(c, carry):
            @pl.when(c < n_chunks - LAG)
            def _():
                gather(c + LAG).start()

            @pl.when(c >= 2)
            def _():
                writeback(c - 2, lax.rem(c, 2)).wait()

            gather(c).wait()
            gb = lax.rem(c, NG)
            tb = lax.rem(c, 2)
            t2d = t_v.at[tb]

            ng = d_embed // 16

            def trans_i(i0, ibc):
                vals = []
                for u in range(UNROLL):
                    i = i0 * UNROLL + u
                    for kk in range(ng):
                        vals.append(g_v[gb, i, pl.ds(16 * kk, 16)])
                for u in range(UNROLL):
                    ivec = ibc + u
                    for kk in range(ng):
                        plsc.store_scatter(
                            t2d, [e_idx[kk], ivec], vals[u * ng + kk]
                        )
                return ibc + UNROLL

            lax.fori_loop(
                0, CHUNK // UNROLL, trans_i,
                jnp.zeros((16,), jnp.int32),
            )
            writeback(c, tb).start()
            return carry

        lax.fori_loop(0, n_chunks, body, 0)
        writeback(n_chunks - 2, lax.rem(n_chunks - 2, 2)).wait()
        writeback(n_chunks - 1, lax.rem(n_chunks - 1, 2)).wait()

    return k


def kernel(x, table):
    n_i, n_j = x.shape
    d_embed = table.shape[1]
    d_pad = 128
    xt = x.T.astype(jnp.int32)
    tpad = jnp.pad(table, ((0, 0), (0, d_pad - d_embed)))
    out3 = _build(n_i, n_j, d_embed, d_pad)(xt, tpad)
    return jnp.transpose(out3, (2, 0, 1))
